# Initial kernel scaffold; baseline (speedup 1.0000x reference)
import jax
import jax.numpy as jnp
from jax.experimental import pallas as pl


def kernel(peptide_x, mhc_x, peptide_emb_w, mhc_emb_w):
    raise NotImplementedError("write your pallas kernel here")


# speedup vs baseline: 1.2737x; 1.2737x over previous
import functools

import jax
import jax.numpy as jnp
from jax import lax
from jax.experimental import pallas as pl
from jax.experimental.pallas import tpu as pltpu
from jax.experimental.pallas import tpu_sc as plsc

VOCAB = 21
EMB = 128
BATCH = 16384
PEP_LEN = 21
MHC_LEN = 34
PEPTIDE_PAD = 3

NC = 2
NS = 16
NW = NC * NS

CH = 128

P_ROWS = BATCH * PEP_LEN
M_ROWS = BATCH * MHC_LEN
PC = P_ROWS // (NW * CH)
MC = M_ROWS // (NW * CH)


def _stream_table(idx_v, table_hbm, out_hbm, rows, gsems, ssems, nchunks, wid):
    wbase = wid * nchunks
    nsup = nchunks // 2

    def gather(c, slot):
        return pltpu.make_async_copy(
            table_hbm.at[idx_v.at[pl.ds(c * CH, CH)]], rows[slot], gsems[slot])

    def scatter(c, slot):
        return pltpu.make_async_copy(
            rows[slot], out_hbm.at[pl.ds((wbase + c) * CH, CH)], ssems[slot])

    gather(0, 0).start()

    def sup(c2, carry):
        a = 2 * c2
        b = a + 1
        gather(a, 0).wait()
        gather(b, 1).start()
        scatter(a, 0).start()
        gather(b, 1).wait()
        scatter(b, 1).start()
        scatter(a, 0).wait()

        @pl.when(c2 + 1 < nsup)
        def _():
            gather(a + 2, 0).start()

        scatter(b, 1).wait()
        return carry

    lax.fori_loop(0, nsup, sup, 0, unroll=False)


def _sc_body(pidx_hbm, midx_hbm, pw_hbm, mw_hbm, pout_hbm, mout_hbm,
             pidx_v, midx_v, rows0, rows1, gsem0, gsem1, ssem0, ssem1):
    wid = lax.axis_index("s") * NC + lax.axis_index("c")

    pltpu.sync_copy(pidx_hbm.at[pl.ds(wid * (PC * CH), PC * CH)], pidx_v)
    pltpu.sync_copy(midx_hbm.at[pl.ds(wid * (MC * CH), MC * CH)], midx_v)

    rows = (rows0, rows1)
    gsems = (gsem0, gsem1)
    ssems = (ssem0, ssem1)
    _stream_table(pidx_v, pw_hbm, pout_hbm, rows, gsems, ssems, PC, wid)
    _stream_table(midx_v, mw_hbm, mout_hbm, rows, gsems, ssems, MC, wid)


_sc_gather = pl.kernel(
    _sc_body,
    out_type=(
        jax.ShapeDtypeStruct((P_ROWS, EMB), jnp.float32),
        jax.ShapeDtypeStruct((M_ROWS, EMB), jnp.float32),
    ),
    mesh=plsc.VectorSubcoreMesh(core_axis_name="c", subcore_axis_name="s"),
    scratch_types=[
        pltpu.VMEM((PC * CH,), jnp.int32),
        pltpu.VMEM((MC * CH,), jnp.int32),
        pltpu.VMEM((CH, EMB), jnp.float32),
        pltpu.VMEM((CH, EMB), jnp.float32),
        pltpu.SemaphoreType.DMA,
        pltpu.SemaphoreType.DMA,
        pltpu.SemaphoreType.DMA,
        pltpu.SemaphoreType.DMA,
    ],
)


def _mask_body(x_ref, o_ref):
    o_ref[...] = x_ref[...] != 0


_mask = pl.pallas_call(
    _mask_body,
    out_shape=jax.ShapeDtypeStruct((BATCH, PEP_LEN - 2 * PEPTIDE_PAD), jnp.bool_),
    grid=(8,),
    in_specs=[pl.BlockSpec((BATCH // 8, PEP_LEN - 2 * PEPTIDE_PAD),
                           lambda i: (i, 0))],
    out_specs=pl.BlockSpec((BATCH // 8, PEP_LEN - 2 * PEPTIDE_PAD),
                           lambda i: (i, 0)),
)


def kernel(peptide_x, mhc_x, peptide_emb_w, mhc_emb_w):
    pidx = peptide_x.astype(jnp.int32).reshape(P_ROWS)
    midx = mhc_x.astype(jnp.int32).reshape(M_ROWS)
    pep_rows, mhc_rows = _sc_gather(pidx, midx, peptide_emb_w, mhc_emb_w)
    pep_emb = pep_rows.reshape(BATCH, PEP_LEN, EMB)
    mhc_emb = mhc_rows.reshape(BATCH, MHC_LEN, EMB)
    masks = _mask(peptide_x[:, PEPTIDE_PAD:PEP_LEN - PEPTIDE_PAD].astype(jnp.int32))
    return pep_emb, mhc_emb, masks
